# baseline (speedup 1.0000x reference)
import functools

import jax
import jax.numpy as jnp
from jax import lax
from jax.experimental import pallas as pl
from jax.experimental.pallas import tpu as pltpu
from jax.experimental.pallas import tpu_sc as plsc

K = 64
B, S, D = 4, 2048, 4096
NCORES = 2
NSUB = 16
ROWS_PER_SUB = (B * S) // (NCORES * NSUB)
CH = 4
NCH = ROWS_PER_SUB // CH
NSLOT = 4
LEAD = 2
PF_PER_SUB = 16
SCATTER_WORKERS = K // PF_PER_SUB

_mesh = plsc.VectorSubcoreMesh(core_axis_name="c", subcore_axis_name="s")


@functools.partial(
    pl.kernel,
    out_type=jax.ShapeDtypeStruct((B * S, D), jnp.float32),
    mesh=_mesh,
    scratch_types=[
        pltpu.VMEM((NSLOT * CH, D), jnp.float32),
        pltpu.VMEM((16,), jnp.int32),
        pltpu.VMEM((16,), jnp.int32),
        [pltpu.SemaphoreType.DMA] * NSLOT,
        [pltpu.SemaphoreType.DMA] * NSLOT,
        pltpu.SemaphoreType.DMA,
    ],
)
def _postfix_kernel(x_hbm, seq_hbm, pf_hbm, out_hbm,
                    buf, seq_v, idx_v, gsem, ssem, psem):
    c = lax.axis_index("c")
    s = lax.axis_index("s")
    wid = c * NSUB + s
    row0 = wid * ROWS_PER_SUB
    slot = [buf.at[pl.ds(u * CH, CH)] for u in range(NSLOT)]

    def rows(i):
        return pl.ds(row0 + i * CH, CH)

    def gstart(u, ci):
        pltpu.make_async_copy(x_hbm.at[rows(ci)], slot[u], gsem[u]).start()

    def gwait(u, ci):
        pltpu.make_async_copy(x_hbm.at[rows(ci)], slot[u], gsem[u]).wait()

    def sstart(u, ci):
        pltpu.make_async_copy(slot[u], out_hbm.at[rows(ci)], ssem[u]).start()

    def swait(u, ci):
        pltpu.make_async_copy(slot[u], out_hbm.at[rows(ci)], ssem[u]).wait()

    sm = s % (NSUB // 2)
    is_worker = sm < SCATTER_WORKERS
    b = 2 * c + s // (NSUB // 2)
    j = sm
    pf_stage = buf.at[pl.ds(0, PF_PER_SUB)]
    pf_in = pltpu.make_async_copy(
        pf_hbm.at[pl.ds(j * PF_PER_SUB, PF_PER_SUB)], pf_stage, psem)

    pltpu.sync_copy(seq_hbm.at[b], seq_v)

    for cc in range(LEAD):
        gstart(cc % NSLOT, cc)
    for cc in range(NCH):
        gwait(cc % NSLOT, cc)
        sstart(cc % NSLOT, cc)
        nc = cc + LEAD
        if nc < NCH:
            un = nc % NSLOT
            if nc - NSLOT >= 0:
                swait(un, nc - NSLOT)
            gstart(un, nc)
    for cc in range(NCH - NSLOT, NCH):
        swait(cc % NSLOT, cc)

    @pl.when(is_worker)
    def _():
        pf_in.start()

    plsc.subcore_barrier()

    @pl.when(is_worker)
    def _():
        pf_in.wait()
        lane = lax.iota(jnp.int32, 16)
        idx_v[...] = seq_v[...] + b * S + j * PF_PER_SUB + lane
        pltpu.async_copy(pf_stage, out_hbm.at[idx_v], psem).wait()


def kernel(crossattn_emb, crossattn_seqlens, postfix_embeds):
    x2d = crossattn_emb.reshape(B * S, D)
    seq_bcast = jnp.broadcast_to(
        crossattn_seqlens.astype(jnp.int32)[:, None], (B, 16))
    out2d = _postfix_kernel(x2d, seq_bcast, postfix_embeds)
    return out2d.reshape(B, S, D)


# speedup vs baseline: 1.0284x; 1.0284x over previous
import functools

import jax
import jax.numpy as jnp
from jax import lax
from jax.experimental import pallas as pl
from jax.experimental.pallas import tpu as pltpu
from jax.experimental.pallas import tpu_sc as plsc

K = 64
B, S, D = 4, 2048, 4096
NCORES = 2
NSUB = 16
ROWS_PER_SUB = (B * S) // (NCORES * NSUB)
CH = 4
NCH = ROWS_PER_SUB // CH
NSLOT = 4
LEAD = 2
PF_PER_SUB = 16
SCATTER_WORKERS = K // PF_PER_SUB

_mesh = plsc.VectorSubcoreMesh(core_axis_name="c", subcore_axis_name="s")


@functools.partial(
    pl.kernel,
    out_type=jax.ShapeDtypeStruct((B * S, D), jnp.float32),
    mesh=_mesh,
    scratch_types=[
        pltpu.VMEM((NSLOT * CH, D), jnp.float32),
        pltpu.VMEM((16,), jnp.int32),
        pltpu.VMEM((16,), jnp.int32),
        [pltpu.SemaphoreType.DMA] * NSLOT,
        [pltpu.SemaphoreType.DMA] * NSLOT,
        pltpu.SemaphoreType.DMA,
    ],
)
def _postfix_kernel(x_hbm, seq_hbm, pf_hbm, out_hbm,
                    buf, seq_v, idx_v, gsem, ssem, psem):
    c = lax.axis_index("c")
    s = lax.axis_index("s")
    wid = c * NSUB + s
    row0 = wid * ROWS_PER_SUB
    slot = [buf.at[pl.ds(u * CH, CH)] for u in range(NSLOT)]

    def rows(i):
        return pl.ds(row0 + i * CH, CH)

    def gstart(u, ci):
        pltpu.make_async_copy(x_hbm.at[rows(ci)], slot[u], gsem[u]).start()

    def gwait(u, ci):
        pltpu.make_async_copy(x_hbm.at[rows(ci)], slot[u], gsem[u]).wait()

    def sstart(u, ci):
        pltpu.make_async_copy(slot[u], out_hbm.at[rows(ci)], ssem[u]).start()

    def swait(u, ci):
        pltpu.make_async_copy(slot[u], out_hbm.at[rows(ci)], ssem[u]).wait()

    sm = s % (NSUB // 2)
    is_worker = sm < SCATTER_WORKERS
    b = 2 * c + s // (NSUB // 2)
    j = sm
    pf_stage = buf.at[pl.ds(0, PF_PER_SUB)]
    pf_in = pltpu.make_async_copy(
        pf_hbm.at[pl.ds(j * PF_PER_SUB, PF_PER_SUB)], pf_stage, psem)

    pltpu.sync_copy(seq_hbm.at[b], seq_v)

    gstart(0, 0)
    gstart(1, 1)
    gwait(0, 0); sstart(0, 0); gstart(2, 2)
    gwait(1, 1); sstart(1, 1); gstart(3, 3)
    gwait(2, 2); sstart(2, 2); swait(0, 0); gstart(0, 4)
    gwait(3, 3); sstart(3, 3); swait(1, 1); gstart(1, 5)

    @pl.loop(1, NCH // NSLOT - 1)
    def _pipe(i):
        base = NSLOT * i
        gwait(0, base + 0); sstart(0, base + 0); swait(2, base - 2); gstart(2, base + 2)
        gwait(1, base + 1); sstart(1, base + 1); swait(3, base - 1); gstart(3, base + 3)
        gwait(2, base + 2); sstart(2, base + 2); swait(0, base + 0); gstart(0, base + 4)
        gwait(3, base + 3); sstart(3, base + 3); swait(1, base + 1); gstart(1, base + 5)

    base = NCH - NSLOT
    gwait(0, base + 0); sstart(0, base + 0); swait(2, base - 2); gstart(2, base + 2)
    gwait(1, base + 1); sstart(1, base + 1); swait(3, base - 1); gstart(3, base + 3)
    gwait(2, base + 2); sstart(2, base + 2); swait(0, base + 0)
    gwait(3, base + 3); sstart(3, base + 3); swait(1, base + 1)
    swait(2, base + 2)
    swait(3, base + 3)

    @pl.when(is_worker)
    def _():
        pf_in.start()

    plsc.subcore_barrier()

    @pl.when(is_worker)
    def _():
        pf_in.wait()
        lane = lax.iota(jnp.int32, 16)
        idx_v[...] = seq_v[...] + b * S + j * PF_PER_SUB + lane
        pltpu.async_copy(pf_stage, out_hbm.at[idx_v], psem).wait()


def kernel(crossattn_emb, crossattn_seqlens, postfix_embeds):
    x2d = crossattn_emb.reshape(B * S, D)
    seq_bcast = jnp.broadcast_to(
        crossattn_seqlens.astype(jnp.int32)[:, None], (B, 16))
    out2d = _postfix_kernel(x2d, seq_bcast, postfix_embeds)
    return out2d.reshape(B, S, D)
